# Initial kernel scaffold; baseline (speedup 1.0000x reference)
import jax
import jax.numpy as jnp
from jax.experimental import pallas as pl


def kernel(x, W1, W2, W3, W4, W5, g1, b1, g2, b2, g3, b3, g4, b4):
    raise NotImplementedError("write your pallas kernel here")


# speedup vs baseline: 1.0003x; 1.0003x over previous
import jax
import jax.numpy as jnp
from jax.experimental import pallas as pl

EPS = 1e-5


def _knn(x, k):
    xt = jnp.transpose(x, (0, 2, 1))
    sq = jnp.sum(xt * xt, axis=-1)
    inner = jnp.einsum('bnc,bmc->bnm', xt, xt)
    dist = sq[:, :, None] - 2.0 * inner + sq[:, None, :]
    _, idx = jax.lax.top_k(-dist, k)
    return idx


def _get_graph_feature(x, k):
    B, C, N = x.shape
    idx = _knn(x, k)
    xt = jnp.transpose(x, (0, 2, 1))
    feature = jax.vmap(lambda a, i: a[i])(xt, idx)
    xc = jnp.broadcast_to(xt[:, :, None, :], (B, N, k, C))
    feat = jnp.concatenate([feature - xc, xc], axis=-1)
    return jnp.transpose(feat, (0, 3, 1, 2))


def _bn2d(x, g, b):
    m = jnp.mean(x, axis=(0, 2, 3), keepdims=True)
    v = jnp.mean((x - m) ** 2, axis=(0, 2, 3), keepdims=True)
    return (x - m) / jnp.sqrt(v + EPS) * g[None, :, None, None] + b[None, :, None, None]


def _conv_block(x, W, g, b):
    y = jnp.einsum('oi,binm->bonm', W, x)
    y = _bn2d(y, g, b)
    return jnp.where(y >= 0, y, 0.2 * y)


def _proj_kernel(xc_ref, w_ref, o_ref):
    o_ref[0] = jnp.dot(w_ref[...], xc_ref[0],
                       preferred_element_type=jnp.float32)


def kernel(x, W1, W2, W3, W4, W5, g1, b1, g2, b2, g3, b3, g4, b4):
    k = 40
    f = _get_graph_feature(x, k)
    x1 = jnp.max(_conv_block(f, W1, g1, b1), axis=-1)
    f = _get_graph_feature(x1, k)
    x2 = jnp.max(_conv_block(f, W2, g2, b2), axis=-1) + x1
    f = _get_graph_feature(x2, k)
    x3 = jnp.max(_conv_block(f, W3, g3, b3), axis=-1) + x2
    f = _get_graph_feature(x3, k)
    x4 = jnp.max(_conv_block(f, W4, g4, b4), axis=-1) + x3
    xc = jnp.concatenate((x1, x2, x3, x4), axis=1)
    B, C2, N = xc.shape
    out = pl.pallas_call(
        _proj_kernel,
        grid=(B,),
        in_specs=[
            pl.BlockSpec((1, C2, N), lambda b: (b, 0, 0)),
            pl.BlockSpec((C2, C2), lambda b: (0, 0)),
        ],
        out_specs=pl.BlockSpec((1, C2, N), lambda b: (b, 0, 0)),
        out_shape=jax.ShapeDtypeStruct((B, C2, N), jnp.float32),
    )(xc, W5)
    return out


# speedup vs baseline: 11.7515x; 11.7485x over previous
import functools

import jax
import jax.numpy as jnp
from jax import lax
from jax.experimental import pallas as pl
from jax.experimental.pallas import tpu as pltpu

EPS = 1e-5
KNN = 40
NEG = -3.0e38


def _select_mask(xt):
    N = xt.shape[1]
    xb = xt.astype(jnp.bfloat16)
    inner = lax.dot_general(xb, xb, (((0,), (0,)), ((), ())),
                            preferred_element_type=jnp.float32)
    sq = jnp.sum(xt * xt, axis=0)
    D = (sq[:, None] - 2.0 * inner) + sq[None, :]

    bits = lax.bitcast_convert_type(D, jnp.int32)
    flipped = jnp.where(bits >= 0, bits ^ jnp.int32(-2**31), ~bits)
    ukey = lax.bitcast_convert_type(flipped, jnp.uint32)

    p = jnp.zeros((N, 1), jnp.uint32)
    for bit in range(31, -1, -1):
        t_c = p | jnp.uint32((1 << bit) - 1)
        cnt = jnp.sum((ukey <= t_c).astype(jnp.int32), axis=1)
        ge = (cnt >= KNN).reshape(N, 1)
        p = jnp.where(ge, p, p | jnp.uint32(1 << bit))
    return (ukey <= p).astype(jnp.float32)


def _edge_compute(C, xs_ref, mf_ref, wbd_ref, ypm_ref):
    N = xs_ref.shape[1]
    xt = xs_ref[...]
    xrep = jnp.concatenate([xt] * 8, axis=1)

    def g_body(g, carry):
        s1, s2 = carry
        goff = pl.multiple_of(128 * g, 128)
        xsl = xs_ref[:, pl.ds(goff, 128)]
        cols = []
        for sg in range(4):
            fbands = []
            for b in range(4):
                o0 = 32 * sg + 8 * b
                sel = jnp.concatenate(
                    [jnp.broadcast_to(xsl[:, o0 + t:o0 + t + 1], (C, N))
                     for t in range(8)], axis=1)
                d_b = (xrep - sel).astype(jnp.bfloat16)
                c_b = sel.astype(jnp.bfloat16)
                fbands.append(jnp.concatenate([d_b, c_b], axis=0))
            F4 = jnp.concatenate(fbands, axis=0)
            y4 = lax.dot_general(wbd_ref[...], F4, (((1,), (0,)), ((), ())),
                                 preferred_element_type=jnp.float32)
            for b in range(4):
                yb = y4[32 * b:32 * b + 32, :]
                mo = pl.multiple_of(128 * g + 32 * sg + 8 * b, 8)
                mrows = mf_ref[pl.ds(mo, 8), :]
                mxs = []
                for t in range(8):
                    seg = yb[:, N * t:N * (t + 1)]
                    mr = mrows[t:t + 1, :] > 0.0
                    mxs.append(jnp.max(jnp.where(mr, seg, NEG), axis=1))
                    mz = jnp.where(mr, seg, 0.0)
                    s1 = s1 + jnp.sum(mz, axis=1)
                    s2 = s2 + jnp.sum(mz * seg, axis=1)
                cols.append(jnp.stack(mxs, axis=1))
        ypm_ref[0, :, pl.ds(goff, 128)] = jnp.concatenate(cols, axis=1)
        return (s1, s2)

    z32 = jnp.zeros((32,), jnp.float32)
    return lax.fori_loop(0, N // 128, g_body, (z32, z32))


def _acc_out(ref, val):
    @pl.when(pl.program_id(0) == 0)
    def _():
        ref[...] = val

    @pl.when(pl.program_id(0) != 0)
    def _():
        ref[...] += val


def _layer_body(C, is_first, has_prev, *refs):
    if is_first:
        x_ref = refs[0]
        refs = refs[1:]
    else:
        ypmp_ref, sc_ref, bi_ref = refs[:3]
        refs = refs[3:]
        if has_prev:
            xprev_ref = refs[0]
            refs = refs[1:]
    wbd_ref = refs[0]
    refs = refs[1:]
    if not is_first:
        xout_ref = refs[0]
        refs = refs[1:]
    ypm_ref, s1_ref, s2_ref, xs_ref, mf_ref = refs

    if is_first:
        xt = x_ref[0]
    else:
        z = ypmp_ref[0] * sc_ref[...] + bi_ref[...]
        xt = jnp.where(z >= 0, z, 0.2 * z)
        if has_prev:
            xt = xt + xprev_ref[0]
        xout_ref[0] = xt
    xs_ref[...] = xt
    mf_ref[...] = _select_mask(xt)
    s1, s2 = _edge_compute(C, xs_ref, mf_ref, wbd_ref, ypm_ref)
    _acc_out(s1_ref, s1.reshape(1, 32))
    _acc_out(s2_ref, s2.reshape(1, 32))


def _final_body(ypm_ref, sc_ref, bi_ref, x3_ref, x1_ref, x2_ref, w5_ref,
                out_ref):
    z = ypm_ref[0] * sc_ref[...] + bi_ref[...]
    x4 = jnp.where(z >= 0, z, 0.2 * z) + x3_ref[0]
    cat = jnp.concatenate([x1_ref[0], x2_ref[0], x3_ref[0], x4], axis=0)
    out_ref[0] = lax.dot_general(w5_ref[...], cat.astype(jnp.bfloat16),
                                 (((1,), (0,)), ((), ())),
                                 preferred_element_type=jnp.float32)


def _full(shape):
    nd = len(shape)
    return pl.BlockSpec(shape, lambda b: (0,) * nd)


def _batched(shape):
    nd = len(shape)
    return pl.BlockSpec((1,) + shape, lambda b: (b,) + (0,) * nd)


def _blockdiag(W):
    K2 = W.shape[1]
    Z = jnp.zeros((128, 4 * K2), jnp.float32)
    for b in range(4):
        Z = Z.at[32 * b:32 * b + 32, K2 * b:K2 * (b + 1)].set(W)
    return Z.astype(jnp.bfloat16)


def kernel(x, W1, W2, W3, W4, W5, g1, b1, g2, b2, g3, b3, g4, b4):
    B, C0, N = x.shape
    count = B * N * KNN
    f32 = jnp.float32

    def run_layer(C, is_first, has_prev, ins, wbd):
        outs = []
        ospecs = []
        if not is_first:
            outs.append(jax.ShapeDtypeStruct((B, 32, N), f32))
            ospecs.append(_batched((32, N)))
        outs += [jax.ShapeDtypeStruct((B, 32, N), f32),
                 jax.ShapeDtypeStruct((1, 32), f32),
                 jax.ShapeDtypeStruct((1, 32), f32)]
        ospecs += [_batched((32, N)), _full((1, 32)), _full((1, 32))]
        ispecs = []
        for a in ins:
            if a.ndim == 3:
                ispecs.append(_batched(a.shape[1:]))
            else:
                ispecs.append(_full(a.shape))
        ispecs.append(_full(wbd.shape))
        body = functools.partial(_layer_body, C, is_first, has_prev)
        return pl.pallas_call(
            body, grid=(B,),
            in_specs=ispecs, out_specs=ospecs, out_shape=outs,
            scratch_shapes=[
                pltpu.VMEM((C, N), f32),
                pltpu.VMEM((N, N), f32),
            ],
        )(*ins, wbd)

    def stats(s1, s2, g, b):
        m = s1.reshape(32) / count
        var = s2.reshape(32) / count - m * m
        scale = g / jnp.sqrt(var + EPS)
        bias = b - m * scale
        return scale.reshape(32, 1), bias.reshape(32, 1)

    ypm1, s1, s2 = run_layer(C0, True, False, [x], _blockdiag(W1))
    sc1, bi1 = stats(s1, s2, g1, b1)
    x1, ypm2, s1, s2 = run_layer(32, False, False, [ypm1, sc1, bi1],
                                 _blockdiag(W2))
    sc2, bi2 = stats(s1, s2, g2, b2)
    x2, ypm3, s1, s2 = run_layer(32, False, True, [ypm2, sc2, bi2, x1],
                                 _blockdiag(W3))
    sc3, bi3 = stats(s1, s2, g3, b3)
    x3, ypm4, s1, s2 = run_layer(32, False, True, [ypm3, sc3, bi3, x2],
                                 _blockdiag(W4))
    sc4, bi4 = stats(s1, s2, g4, b4)

    out = pl.pallas_call(
        _final_body,
        grid=(B,),
        in_specs=[_batched((32, N)), _full((32, 1)), _full((32, 1)),
                  _batched((32, N)), _batched((32, N)), _batched((32, N)),
                  _full((128, 128))],
        out_specs=_batched((128, N)),
        out_shape=jax.ShapeDtypeStruct((B, 128, N), f32),
    )(ypm4, sc4, bi4, x3, x1, x2, W5.astype(jnp.bfloat16))
    return out
